# Initial kernel scaffold; baseline (speedup 1.0000x reference)
import jax
import jax.numpy as jnp
from jax.experimental import pallas as pl


def kernel(x, edge_index, batch, W1, a1_src, a1_dst, b1, W2, a2_src, a2_dst, b2, Wl, bl):
    raise NotImplementedError("write your pallas kernel here")


# speedup vs baseline: 1.0082x; 1.0082x over previous
import functools

import jax
import jax.numpy as jnp
from jax.experimental import pallas as pl

_N = 10000
_HEADS = 4
_HIDDEN = 256
_NUM_GRAPHS = 128


def _mm_body(x_ref, w_ref, o_ref):
    o_ref[...] = jnp.dot(x_ref[...], w_ref[...],
                         preferred_element_type=jnp.float32)


@functools.partial(jax.jit, static_argnames=())
def _mm(x, w):
    m, k = x.shape
    _, n_out = w.shape
    bm = 1000 if m % 1000 == 0 else m
    return pl.pallas_call(
        _mm_body,
        grid=(m // bm,),
        in_specs=[
            pl.BlockSpec((bm, k), lambda i: (i, 0)),
            pl.BlockSpec((k, n_out), lambda i: (0, 0)),
        ],
        out_specs=pl.BlockSpec((bm, n_out), lambda i: (i, 0)),
        out_shape=jax.ShapeDtypeStruct((m, n_out), jnp.float32),
    )(x, w)


def _attn_proj_weights(a_src, a_dst, out_ch):
    heads = a_src.shape[0]
    eye = jnp.eye(heads, dtype=jnp.float32)
    asrc = (eye[:, None, :] * a_src[:, :, None]).reshape(heads * out_ch, heads)
    adst = (eye[:, None, :] * a_dst[:, :, None]).reshape(heads * out_ch, heads)
    return jnp.concatenate([asrc, adst], axis=1)


def _gat_layer(x, src, dst, W, a_src, a_dst, bias, heads, out_ch, concat):
    n = x.shape[0]
    h = _mm(x, W)
    A = _attn_proj_weights(a_src, a_dst, out_ch)
    al = _mm(h, A)
    alpha_s = al[:, :heads]
    alpha_d = al[:, heads:]
    e = alpha_s[src] + alpha_d[dst]
    e = jax.nn.leaky_relu(e, negative_slope=0.2)
    m = jax.ops.segment_max(e, dst, num_segments=n)
    m = jnp.where(jnp.isfinite(m), m, 0.0)
    ex = jnp.exp(e - m[dst])
    denom = jax.ops.segment_sum(ex, dst, num_segments=n)
    alpha = ex / (denom[dst] + 1e-16)
    msg = h[src].reshape(-1, heads, out_ch) * alpha[:, :, None]
    out = jax.ops.segment_sum(msg, dst, num_segments=n)
    if concat:
        out = out.reshape(n, heads * out_ch)
    else:
        out = jnp.mean(out, axis=1)
    return out + bias[None, :]


def kernel(x, edge_index, batch, W1, a1_src, a1_dst, b1,
           W2, a2_src, a2_dst, b2, Wl, bl):
    src = edge_index[0]
    dst = edge_index[1]
    h = _gat_layer(x, src, dst, W1, a1_src, a1_dst, b1, _HEADS, _HIDDEN, True)
    h = jax.nn.elu(h)
    h = _gat_layer(h, src, dst, W2, a2_src, a2_dst, b2, 1, _HIDDEN, False)
    h = jax.nn.elu(h)
    pooled = jax.ops.segment_sum(h, batch, num_segments=_NUM_GRAPHS)
    return _mm(pooled, Wl) + bl[None, :]


# speedup vs baseline: 1.0122x; 1.0040x over previous
import functools

import jax
import jax.numpy as jnp
from jax.experimental import pallas as pl

_N = 10000
_HEADS = 4
_HIDDEN = 256
_NUM_GRAPHS = 128


def _mm_body(x_ref, w_ref, o_ref):
    o_ref[...] = jnp.dot(x_ref[...], w_ref[...],
                         preferred_element_type=jnp.float32)


@functools.partial(jax.jit, static_argnames=())
def _mm(x, w):
    m, k = x.shape
    _, n_out = w.shape
    bm = 1000 if m % 1000 == 0 else m
    return pl.pallas_call(
        _mm_body,
        grid=(m // bm,),
        in_specs=[
            pl.BlockSpec((bm, k), lambda i: (i, 0)),
            pl.BlockSpec((k, n_out), lambda i: (0, 0)),
        ],
        out_specs=pl.BlockSpec((bm, n_out), lambda i: (i, 0)),
        out_shape=jax.ShapeDtypeStruct((m, n_out), jnp.float32),
    )(x, w)


def _attn_proj_weights(a_src, a_dst, out_ch):
    heads = a_src.shape[0]
    eye = jnp.eye(heads, dtype=jnp.float32)
    asrc = (eye[:, None, :] * a_src[:, :, None]).reshape(heads * out_ch, heads)
    adst = (eye[:, None, :] * a_dst[:, :, None]).reshape(heads * out_ch, heads)
    return jnp.concatenate([asrc, adst], axis=1)


def _gat_layer(x, src, dst, W, a_src, a_dst, bias, heads, out_ch, concat):
    n = x.shape[0]
    d = heads * out_ch
    A = _attn_proj_weights(a_src, a_dst, out_ch)
    Wbig = jnp.concatenate([W, W @ A], axis=1)
    hal = _mm(x, Wbig)
    h = hal[:, :d]
    alpha_s = hal[:, d:d + heads]
    alpha_d = hal[:, d + heads:]
    e = alpha_s[src] + alpha_d[dst]
    e = jax.nn.leaky_relu(e, negative_slope=0.2)
    m = jax.ops.segment_max(e, dst, num_segments=n)
    m = jnp.where(jnp.isfinite(m), m, 0.0)
    ex = jnp.exp(e - m[dst])
    denom = jax.ops.segment_sum(ex, dst, num_segments=n)
    alpha = ex / (denom[dst] + 1e-16)
    msg = h[src].reshape(-1, heads, out_ch) * alpha[:, :, None]
    out = jax.ops.segment_sum(msg, dst, num_segments=n)
    if concat:
        out = out.reshape(n, heads * out_ch)
    else:
        out = jnp.mean(out, axis=1)
    return out + bias[None, :]


def kernel(x, edge_index, batch, W1, a1_src, a1_dst, b1,
           W2, a2_src, a2_dst, b2, Wl, bl):
    src = edge_index[0]
    dst = edge_index[1]
    h = _gat_layer(x, src, dst, W1, a1_src, a1_dst, b1, _HEADS, _HIDDEN, True)
    h = jax.nn.elu(h)
    h = _gat_layer(h, src, dst, W2, a2_src, a2_dst, b2, 1, _HIDDEN, False)
    h = jax.nn.elu(h)
    pooled = jax.ops.segment_sum(h, batch, num_segments=_NUM_GRAPHS)
    return _mm(pooled, Wl) + bl[None, :]
